# baseline (speedup 1.0000x reference)
import jax
import jax.numpy as jnp
from jax import lax
from jax.experimental import pallas as pl
from jax.experimental.pallas import tpu as pltpu
from jax.experimental.pallas import tpu_sc as plsc

_N_COLS = 26
_VOCAB = 100000
_D = 32
_B = 16384
_NC, _NS = 2, 16
_NW = _NC * _NS
_BPW = _B // _NW
_NBB = 8
_NGRP = _BPW // (2 * _NBB)
_LANES = 16
_W = 2 * _LANES


def _body(Xp, table, out, idx1, rows_v, isem, gsem, osem0, osem1):
    wid = lax.axis_index("s") * _NC + lax.axis_index("c")
    lanes = lax.iota(jnp.int32, _LANES)
    off_lo = lanes * _VOCAB
    off_hi = jnp.minimum(lanes + _LANES, _N_COLS - 1) * _VOCAB
    osems = (osem0, osem1)

    @pl.loop(0, _NGRP)
    def _group(g):
        for par in range(2):
            b0 = wid * _BPW + (2 * g + par) * _NBB

            for bl in range(_NBB):
                pltpu.async_copy(Xp.at[b0 + bl],
                                 idx1.at[par, pl.ds(bl * _W, _W)], isem)
            for bl in range(_NBB):
                pltpu.make_async_copy(Xp.at[0],
                                      idx1.at[par, pl.ds(bl * _W, _W)],
                                      isem).wait()

            for bl in range(_NBB):
                lo = pl.ds(bl * _W, _LANES)
                hi = pl.ds(bl * _W + _LANES, _LANES)
                idx1[par, lo] = idx1[par, lo] + off_lo
                idx1[par, hi] = idx1[par, hi] + off_hi

            @pl.when(g > 0)
            def _():
                for bl in range(_NBB):
                    pltpu.make_async_copy(
                        rows_v.at[par, pl.ds(bl * _W, _N_COLS)],
                        out.at[0], osems[par]).wait()

            for h in range(2):
                pltpu.async_copy(
                    table.at[idx1.at[par, pl.ds(h * 128, 128)]],
                    rows_v.at[par, pl.ds(h * 128, 128)], gsem)
            for h in range(2):
                pltpu.make_async_copy(
                    table.at[idx1.at[par, pl.ds(h * 128, 128)]],
                    rows_v.at[par, pl.ds(h * 128, 128)], gsem).wait()

            for bl in range(_NBB):
                pltpu.async_copy(
                    rows_v.at[par, pl.ds(bl * _W, _N_COLS)],
                    out.at[b0 + bl], osems[par])

    for par in range(2):
        for bl in range(_NBB):
            pltpu.make_async_copy(
                rows_v.at[par, pl.ds(bl * _W, _N_COLS)],
                out.at[0], osems[par]).wait()


def kernel(X, table):
    Xp = jnp.pad(X, ((0, 0), (0, _W - _N_COLS)))
    mesh = plsc.VectorSubcoreMesh(
        core_axis_name="c", subcore_axis_name="s",
        num_cores=_NC, num_subcores=_NS)
    scratch = [
        pltpu.VMEM((2, _NBB * _W), jnp.int32),
        pltpu.VMEM((2, _NBB * _W, _D), jnp.float32),
        pltpu.SemaphoreType.DMA,
        pltpu.SemaphoreType.DMA,
        pltpu.SemaphoreType.DMA,
        pltpu.SemaphoreType.DMA,
    ]
    return pl.kernel(
        _body,
        out_type=jax.ShapeDtypeStruct((_B, _N_COLS, _D), jnp.float32),
        mesh=mesh,
        scratch_types=scratch,
        compiler_params=pltpu.CompilerParams(use_tc_tiling_on_sc=False),
    )(Xp, table)


# speedup vs baseline: 2.1825x; 2.1825x over previous
import jax
import jax.numpy as jnp
from jax import lax
from jax.experimental import pallas as pl
from jax.experimental.pallas import tpu as pltpu
from jax.experimental.pallas import tpu_sc as plsc

_N_COLS = 26
_VOCAB = 100000
_D = 32
_B = 16384
_NC, _NS = 2, 16
_NW = _NC * _NS
_BPW = _B // _NW
_NBB = 4
_NSUP = _BPW // (2 * _NBB)
_LANES = 16


def _body(X, table, out, idx1, rows_v, gsem, isem, osem0, osem1):
    wid = lax.axis_index("s") * _NC + lax.axis_index("c")
    lanes = lax.iota(jnp.int32, _LANES)
    off_lo = lanes * _VOCAB
    off_hi = (lanes + _LANES) * _VOCAB
    osems = (osem0, osem1)

    @pl.loop(0, _NSUP)
    def _super(g):
        b0 = wid * _BPW + g * (2 * _NBB)
        for bl in range(2 * _NBB):
            pltpu.async_copy(X.at[b0 + bl], idx1.at[bl], isem)
        for bl in range(2 * _NBB):
            pltpu.make_async_copy(X.at[0], idx1.at[bl], isem).wait()

        for half in range(2):
            bh = b0 + half * _NBB

            for bl in range(_NBB):
                row = half * _NBB + bl
                rv0 = plsc.bitcast(
                    idx1[row, pl.ds(0, _LANES)], jnp.int32) + off_lo
                rv1 = plsc.bitcast(
                    idx1[row, pl.ds(_LANES, _LANES)], jnp.int32) + off_hi
                for lane in range(_LANES):
                    pltpu.async_copy(
                        table.at[rv0[lane]], rows_v.at[half, bl, lane], gsem)
                for lane in range(_N_COLS - _LANES):
                    pltpu.async_copy(
                        table.at[rv1[lane]], rows_v.at[half, bl, _LANES + lane],
                        gsem)

            for bl in range(_NBB):
                pltpu.make_async_copy(
                    out.at[0], rows_v.at[half, bl], gsem).wait()

            @pl.when(g > 0)
            def _():
                pltpu.make_async_copy(
                    rows_v.at[half], out.at[pl.ds(0, _NBB)], osems[half]).wait()

            pltpu.async_copy(rows_v.at[half], out.at[pl.ds(bh, _NBB)],
                             osems[half])

    for half in range(2):
        pltpu.make_async_copy(
            rows_v.at[half], out.at[pl.ds(0, _NBB)], osems[half]).wait()


def kernel(X, table):
    Xp = jax.lax.bitcast_convert_type(
        jnp.pad(X, ((0, 0), (0, 2 * _LANES - _N_COLS))), jnp.float32)
    mesh = plsc.VectorSubcoreMesh(
        core_axis_name="c", subcore_axis_name="s",
        num_cores=_NC, num_subcores=_NS)
    scratch = [
        pltpu.VMEM((2 * _NBB, _D), jnp.float32),
        pltpu.VMEM((2, _NBB, _N_COLS, _D), jnp.float32),
        pltpu.SemaphoreType.DMA,
        pltpu.SemaphoreType.DMA,
        pltpu.SemaphoreType.DMA,
        pltpu.SemaphoreType.DMA,
    ]
    return pl.kernel(
        _body,
        out_type=jax.ShapeDtypeStruct((_B, _N_COLS, _D), jnp.float32),
        mesh=mesh,
        scratch_types=scratch,
        compiler_params=pltpu.CompilerParams(needs_layout_passes=False),
    )(Xp, table)
